# baseline (speedup 1.0000x reference)
import jax
import jax.numpy as jnp
from jax.experimental import pallas as pl
from jax.experimental.pallas import tpu as pltpu
from jax.sharding import Mesh, PartitionSpec as P

try:
    from jax import shard_map as _shard_map

    def _smap(f, mesh, in_specs, out_specs):
        return _shard_map(f, mesh=mesh, in_specs=in_specs,
                          out_specs=out_specs, check_vma=False)
except ImportError:
    from jax.experimental.shard_map import shard_map as _shard_map

    def _smap(f, mesh, in_specs, out_specs):
        return _shard_map(f, mesh=mesh, in_specs=in_specs,
                          out_specs=out_specs, check_rep=False)

_GATE_THRESHOLD = 0.5
_GATE_TEMPERATURE = 1.0


def _gate_kernel(scale_ref, bias_ref, x_ref, w_ref, gh_ref, score_ref, mask_ref):
    gh = jax.lax.dot_general(
        x_ref[...], w_ref[...], (((1,), (1,)), ((), ())),
        precision=jax.lax.Precision.DEFAULT,
        preferred_element_type=jnp.float32,
    )
    gh_ref[...] = gh
    rank = gh.shape[1]
    n_chunks = max(1, rank // 256)
    chunk = rank // n_chunks
    ones = jnp.ones((chunk, 8), dtype=jnp.bfloat16)
    col = None
    for c in range(n_chunks):
        ghb = gh[:, c * chunk:(c + 1) * chunk].astype(jnp.bfloat16)
        sq = ghb * ghb
        part = jax.lax.dot_general(
            sq, ones, (((1,), (0,)), ((), ())),
            preferred_element_type=jnp.float32,
        )
        col = part if col is None else col + part
    col = col[:, 0:1]
    score = jnp.sqrt(col) * scale_ref[0, 0] - bias_ref[0, 0]
    keep = score >= (_GATE_THRESHOLD / _GATE_TEMPERATURE)
    score_ref[...] = jnp.where(keep, score, -jnp.inf)
    mask_ref[...] = keep.astype(jnp.float32)


def _gate_shard(scale2, bias2, x_flat, w):
    hidden = x_flat.shape[1]
    rank = w.shape[0]
    m = x_flat.shape[0]
    m_blk = 1024 if m % 1024 == 0 else m
    grid = m // m_blk
    return pl.pallas_call(
        _gate_kernel,
        grid=(grid,),
        in_specs=[
            pl.BlockSpec(memory_space=pltpu.SMEM),
            pl.BlockSpec(memory_space=pltpu.SMEM),
            pl.BlockSpec((m_blk, hidden), lambda i: (i, 0)),
            pl.BlockSpec((rank, hidden), lambda i: (0, 0)),
        ],
        out_specs=[
            pl.BlockSpec((m_blk, rank), lambda i: (i, 0)),
            pl.BlockSpec((m_blk, 1), lambda i: (i, 0)),
            pl.BlockSpec((m_blk, 1), lambda i: (i, 0)),
        ],
        out_shape=[
            jax.ShapeDtypeStruct((m, rank), jnp.float32),
            jax.ShapeDtypeStruct((m, 1), jnp.float32),
            jax.ShapeDtypeStruct((m, 1), jnp.float32),
        ],
        compiler_params=pltpu.CompilerParams(
            dimension_semantics=("arbitrary",),
            vmem_limit_bytes=64 * 1024 * 1024,
        ),
    )(scale2, bias2, x_flat, w)


def kernel(x, W_A, gate_scale, gate_bias):
    orig_shape = x.shape
    hidden = x.shape[-1]
    x_flat = x.reshape(-1, hidden)
    m = x_flat.shape[0]
    scale2 = gate_scale.reshape(1, 1)
    bias2 = gate_bias.reshape(1, 1)

    devs = jax.devices()
    n_shards = 2 if (len(devs) >= 2 and m % 2048 == 0) else 1
    if n_shards > 1:
        mesh = Mesh(devs[:n_shards], ("d",))
        fn = _smap(
            lambda s, b, xf, w: _gate_shard(s, b, xf, w),
            mesh,
            (P(None, None), P(None, None), P("d", None), P(None, None)),
            (P("d", None), P("d", None), P("d", None)),
        )
        gh, score_full, mask_f = fn(scale2, bias2, x_flat, W_A)
    else:
        gh, score_full, mask_f = _gate_shard(scale2, bias2, x_flat, W_A)

    gate_mask_full = mask_f.astype(bool).reshape(orig_shape[:-1])
    gate_score_full = score_full.reshape(orig_shape[:-1])
    return (gate_mask_full, gate_score_full, gh)


# speedup vs baseline: 3.5966x; 3.5966x over previous
import jax
import jax.numpy as jnp
from jax.experimental import pallas as pl
from jax.experimental.pallas import tpu as pltpu

_GATE_THRESHOLD = 0.5
_GATE_TEMPERATURE = 1.0
_N_STREAMS = 8


def _gate_kernel(scale_ref, bias_ref, *refs):
    x_refs = refs[:_N_STREAMS]
    w_ref = refs[_N_STREAMS]
    gh_ref, score_ref, mask_ref = refs[_N_STREAMS + 1:]
    kc = x_refs[0].shape[1]
    parts = []
    for c in range(_N_STREAMS):
        parts.append(jax.lax.dot_general(
            x_refs[c][...], w_ref[:, c * kc:(c + 1) * kc],
            (((1,), (1,)), ((), ())),
            precision=jax.lax.Precision.DEFAULT,
            preferred_element_type=jnp.float32,
        ))
    while len(parts) > 1:
        parts = [parts[i] + parts[i + 1] for i in range(0, len(parts), 2)]
    gh = parts[0]
    gh_ref[...] = gh
    rank = gh.shape[1]
    n_chunks = max(1, rank // 256)
    chunk = rank // n_chunks
    ones = jnp.ones((chunk, 8), dtype=jnp.bfloat16)
    col = None
    for c in range(n_chunks):
        ghb = gh[:, c * chunk:(c + 1) * chunk].astype(jnp.bfloat16)
        sq = ghb * ghb
        part = jax.lax.dot_general(
            sq, ones, (((1,), (0,)), ((), ())),
            preferred_element_type=jnp.float32,
        )
        col = part if col is None else col + part
    col = col[:, 0:1]
    score = jnp.sqrt(col) * scale_ref[0, 0] - bias_ref[0, 0]
    keep = score >= (_GATE_THRESHOLD / _GATE_TEMPERATURE)
    score_ref[...] = jnp.where(keep, score, -jnp.inf)
    mask_ref[...] = keep.astype(jnp.float32)


def kernel(x, W_A, gate_scale, gate_bias):
    orig_shape = x.shape
    hidden = x.shape[-1]
    rank = W_A.shape[0]
    x_flat = x.reshape(-1, hidden)
    m = x_flat.shape[0]
    m_blk = 1024 if m % 1024 == 0 else m
    n_tiles = m // m_blk
    kc = hidden // _N_STREAMS

    scale2 = gate_scale.reshape(1, 1)
    bias2 = gate_bias.reshape(1, 1)

    def _mk_xspec(c):
        return pl.BlockSpec((m_blk, kc), lambda i, c=c: (i, c))

    gh, score_full, mask_f = pl.pallas_call(
        _gate_kernel,
        grid=(n_tiles,),
        in_specs=[
            pl.BlockSpec(memory_space=pltpu.SMEM),
            pl.BlockSpec(memory_space=pltpu.SMEM),
        ] + [_mk_xspec(c) for c in range(_N_STREAMS)] + [
            pl.BlockSpec((rank, hidden), lambda i: (0, 0)),
        ],
        out_specs=[
            pl.BlockSpec((m_blk, rank), lambda i: (i, 0)),
            pl.BlockSpec((m_blk, 1), lambda i: (i, 0)),
            pl.BlockSpec((m_blk, 1), lambda i: (i, 0)),
        ],
        out_shape=[
            jax.ShapeDtypeStruct((m, rank), jnp.float32),
            jax.ShapeDtypeStruct((m, 1), jnp.float32),
            jax.ShapeDtypeStruct((m, 1), jnp.float32),
        ],
        compiler_params=pltpu.CompilerParams(
            dimension_semantics=("arbitrary",),
            vmem_limit_bytes=64 * 1024 * 1024,
        ),
    )(scale2, bias2, *([x_flat] * _N_STREAMS), W_A)

    gate_mask_full = mask_f.astype(bool).reshape(orig_shape[:-1])
    gate_score_full = score_full.reshape(orig_shape[:-1])
    return (gate_mask_full, gate_score_full, gh)


# speedup vs baseline: 3.6902x; 1.0260x over previous
import jax
import jax.numpy as jnp
from jax.experimental import pallas as pl
from jax.experimental.pallas import tpu as pltpu

_GATE_THRESHOLD = 0.5
_GATE_TEMPERATURE = 1.0


def _gate_kernel(scale_ref, bias_ref, x_ref, w_ref, gh_ref, score_ref, mask_ref):
    gh = jax.lax.dot_general(
        x_ref[...], w_ref[...], (((1,), (1,)), ((), ())),
        precision=jax.lax.Precision.DEFAULT,
        preferred_element_type=jnp.float32,
    )
    gh_ref[...] = gh
    rank = gh.shape[1]
    n_chunks = max(1, rank // 256)
    chunk = rank // n_chunks
    ones = jnp.ones((chunk, 8), dtype=jnp.bfloat16)
    col = None
    for c in range(n_chunks):
        ghb = gh[:, c * chunk:(c + 1) * chunk].astype(jnp.bfloat16)
        sq = ghb * ghb
        part = jax.lax.dot_general(
            sq, ones, (((1,), (0,)), ((), ())),
            preferred_element_type=jnp.float32,
        )
        col = part if col is None else col + part
    col = col[:, 0:1]
    score = jnp.sqrt(col) * scale_ref[0, 0] - bias_ref[0, 0]
    keep = score >= (_GATE_THRESHOLD / _GATE_TEMPERATURE)
    score_ref[...] = jnp.where(keep, score, -jnp.inf)
    mask_ref[...] = keep.astype(jnp.float32)


def kernel(x, W_A, gate_scale, gate_bias):
    orig_shape = x.shape
    hidden = x.shape[-1]
    rank = W_A.shape[0]
    x_flat = x.reshape(-1, hidden)
    m = x_flat.shape[0]
    m_blk = 1024 if m % 1024 == 0 else m
    grid = m // m_blk

    scale2 = gate_scale.reshape(1, 1)
    bias2 = gate_bias.reshape(1, 1)

    gh, score_full, mask_f = pl.pallas_call(
        _gate_kernel,
        grid=(grid,),
        in_specs=[
            pl.BlockSpec(memory_space=pltpu.SMEM),
            pl.BlockSpec(memory_space=pltpu.SMEM),
            pl.BlockSpec((m_blk, hidden), lambda i: (i, 0)),
            pl.BlockSpec((rank, hidden), lambda i: (0, 0)),
        ],
        out_specs=[
            pl.BlockSpec((m_blk, rank), lambda i: (i, 0)),
            pl.BlockSpec((m_blk, 1), lambda i: (i, 0)),
            pl.BlockSpec((m_blk, 1), lambda i: (i, 0)),
        ],
        out_shape=[
            jax.ShapeDtypeStruct((m, rank), jnp.float32),
            jax.ShapeDtypeStruct((m, 1), jnp.float32),
            jax.ShapeDtypeStruct((m, 1), jnp.float32),
        ],
        compiler_params=pltpu.CompilerParams(
            dimension_semantics=("arbitrary",),
            vmem_limit_bytes=64 * 1024 * 1024,
        ),
    )(scale2, bias2, x_flat, W_A)

    gate_mask_full = mask_f.astype(bool).reshape(orig_shape[:-1])
    gate_score_full = score_full.reshape(orig_shape[:-1])
    return (gate_mask_full, gate_score_full, gh)
